# baseline (speedup 1.0000x reference)
import functools

import jax
import jax.numpy as jnp
from jax.experimental import pallas as pl
from jax.experimental.pallas import tpu as pltpu


def _round_up(x, m):
    return ((x + m - 1) // m) * m


def _lstm_fused_kernel(emb_ref, wcat_ref, bias_ref, wout_ref, bout_ref,
                       out_ref, hx_scr, c_scr, m_scr):
    t_blk_idx = pl.program_id(1)
    Hp = c_scr.shape[1]
    T_BLK = emb_ref.shape[0]

    @pl.when(t_blk_idx == 0)
    def _init():
        hx_scr[...] = jnp.zeros_like(hx_scr)
        c_scr[...] = jnp.zeros_like(c_scr)
        m_scr[...] = jnp.full(m_scr.shape, -jnp.inf, m_scr.dtype)

    wcat = wcat_ref[...]
    bias = bias_ref[...]

    for t in range(T_BLK):
        hx_scr[:, Hp:] = emb_ref[t]
        gates = jnp.dot(hx_scr[...], wcat,
                        preferred_element_type=jnp.float32) + bias
        i = jax.nn.sigmoid(gates[:, 0 * Hp:1 * Hp])
        f = jax.nn.sigmoid(gates[:, 1 * Hp:2 * Hp])
        g = jnp.tanh(gates[:, 2 * Hp:3 * Hp])
        o = jax.nn.sigmoid(gates[:, 3 * Hp:4 * Hp])
        c_new = f * c_scr[...] + i * g
        h_new = o * jnp.tanh(c_new)
        c_scr[...] = c_new
        m_scr[...] = jnp.maximum(m_scr[...], h_new)
        hx_scr[:, :Hp] = h_new.astype(hx_scr.dtype)

    @pl.when(t_blk_idx == pl.num_programs(1) - 1)
    def _finalize():
        out_ref[...] = (jnp.dot(m_scr[...].astype(wout_ref.dtype),
                                wout_ref[...],
                                preferred_element_type=jnp.float32)
                        + bout_ref[...])


@functools.partial(jax.jit, static_argnames=("t_blk",))
def _forward(input_seq, emb_table, w_ih, w_hh, b_ih, b_hh, w_out, b_out,
             *, t_blk=16):
    B, T = input_seq.shape
    E = emb_table.shape[1]
    H = w_hh.shape[1]
    C = w_out.shape[0]

    Bp = _round_up(B, 16)
    Bb = Bp // 2
    Hp = _round_up(H, 128)
    Cp = _round_up(C, 128)
    K = Hp + E

    def pad_gate_cols(w_t):
        k = w_t.shape[0]
        w4 = w_t.reshape(k, 4, H)
        w4 = jnp.pad(w4, ((0, 0), (0, 0), (0, Hp - H)))
        return w4.reshape(k, 4 * Hp)

    whh_p = jnp.pad(pad_gate_cols(w_hh.T), ((0, Hp - H), (0, 0)))
    wih_p = pad_gate_cols(w_ih.T)
    wcat = jnp.concatenate([whh_p, wih_p], axis=0).astype(jnp.bfloat16)
    bias = jnp.pad((b_ih + b_hh).astype(jnp.float32).reshape(4, H),
                   ((0, 0), (0, Hp - H))).reshape(1, 4 * Hp)
    wout_p = jnp.pad(w_out.T, ((0, Hp - H), (0, Cp - C))).astype(jnp.bfloat16)
    bout_p = jnp.pad(b_out.astype(jnp.float32), (0, Cp - C)).reshape(1, Cp)

    emb_tm = jnp.take(emb_table, input_seq.T, axis=0).astype(jnp.bfloat16)
    emb_tm = jnp.pad(emb_tm, ((0, 0), (0, Bp - B), (0, 0)))

    while T % t_blk:
        t_blk //= 2
    grid = (2, T // t_blk)

    out = pl.pallas_call(
        _lstm_fused_kernel,
        out_shape=jax.ShapeDtypeStruct((Bp, Cp), jnp.float32),
        grid_spec=pltpu.PrefetchScalarGridSpec(
            num_scalar_prefetch=0,
            grid=grid,
            in_specs=[
                pl.BlockSpec((t_blk, Bb, E), lambda b, t: (t, b, 0)),
                pl.BlockSpec((K, 4 * Hp), lambda b, t: (0, 0)),
                pl.BlockSpec((1, 4 * Hp), lambda b, t: (0, 0)),
                pl.BlockSpec((Hp, Cp), lambda b, t: (0, 0)),
                pl.BlockSpec((1, Cp), lambda b, t: (0, 0)),
            ],
            out_specs=pl.BlockSpec((Bb, Cp), lambda b, t: (b, 0)),
            scratch_shapes=[
                pltpu.VMEM((Bb, K), jnp.bfloat16),
                pltpu.VMEM((Bb, Hp), jnp.float32),
                pltpu.VMEM((Bb, Hp), jnp.float32),
            ],
        ),
        compiler_params=pltpu.CompilerParams(
            dimension_semantics=("parallel", "arbitrary"),
            vmem_limit_bytes=100 * 1024 * 1024,
        ),
    )(emb_tm, wcat, bias, wout_p, bout_p)

    return out[:B, :C]


def kernel(input_seq, emb_table, w_ih, w_hh, b_ih, b_hh, w_out, b_out):
    return _forward(input_seq, emb_table, w_ih, w_hh, b_ih, b_hh,
                    w_out, b_out, t_blk=16)


# speedup vs baseline: 1.1958x; 1.1958x over previous
import functools

import jax
import jax.numpy as jnp
from jax.experimental import pallas as pl
from jax.experimental.pallas import tpu as pltpu


def _round_up(x, m):
    return ((x + m - 1) // m) * m


def _lstm_fused_kernel(emb_ref, wcat_ref, bias_ref, wout_ref, bout_ref,
                       out_ref,
                       hx0_scr, hx1_scr, c0_scr, c1_scr, m0_scr, m1_scr):
    t_blk_idx = pl.program_id(0)
    Hp = c0_scr.shape[1]
    Bb = c0_scr.shape[0]
    T_BLK = emb_ref.shape[0]

    @pl.when(t_blk_idx == 0)
    def _init():
        for scr in (hx0_scr, hx1_scr, c0_scr, c1_scr):
            scr[...] = jnp.zeros_like(scr)
        for scr in (m0_scr, m1_scr):
            scr[...] = jnp.full(scr.shape, -jnp.inf, scr.dtype)

    wcat = wcat_ref[...]
    bias = bias_ref[...]

    def step(t, half, hx_scr, c_scr, m_scr):
        hx_scr[:, Hp:] = emb_ref[t, half * Bb:(half + 1) * Bb]
        gp = jnp.dot(hx_scr[...], wcat,
                     preferred_element_type=jnp.float32) + bias
        ti = jnp.tanh(gp[:, 0 * Hp:1 * Hp])
        tf = jnp.tanh(gp[:, 1 * Hp:2 * Hp])
        tg = jnp.tanh(gp[:, 2 * Hp:3 * Hp])
        to = jnp.tanh(gp[:, 3 * Hp:4 * Hp])
        c = c_scr[...]
        c_new = 0.5 * ((c + tg) + (tf * c + ti * tg))
        tc = jnp.tanh(c_new)
        h_new = 0.5 * (tc + to * tc)
        c_scr[...] = c_new
        m_scr[...] = jnp.maximum(m_scr[...], h_new)
        hx_scr[:, :Hp] = h_new.astype(hx_scr.dtype)

    for t in range(T_BLK):
        step(t, 0, hx0_scr, c0_scr, m0_scr)
        step(t, 1, hx1_scr, c1_scr, m1_scr)

    @pl.when(t_blk_idx == pl.num_programs(0) - 1)
    def _finalize():
        wout = wout_ref[...]
        out_ref[0:Bb] = (jnp.dot(m0_scr[...].astype(wout.dtype), wout,
                                 preferred_element_type=jnp.float32)
                         + bout_ref[...])
        out_ref[Bb:2 * Bb] = (jnp.dot(m1_scr[...].astype(wout.dtype), wout,
                                      preferred_element_type=jnp.float32)
                              + bout_ref[...])


@functools.partial(jax.jit, static_argnames=("t_blk",))
def _forward(input_seq, emb_table, w_ih, w_hh, b_ih, b_hh, w_out, b_out,
             *, t_blk=16):
    B, T = input_seq.shape
    E = emb_table.shape[1]
    H = w_hh.shape[1]
    C = w_out.shape[0]

    Bp = _round_up(B, 16)
    Bb = Bp // 2
    Hp = _round_up(H, 128)
    Cp = _round_up(C, 128)
    K = Hp + E

    def pad_gate_cols(w_t):
        k = w_t.shape[0]
        w4 = w_t.reshape(k, 4, H)
        w4 = jnp.pad(w4, ((0, 0), (0, 0), (0, Hp - H)))
        return w4.reshape(k, 4 * Hp)

    whh_p = jnp.pad(pad_gate_cols(w_hh.T), ((0, Hp - H), (0, 0)))
    wih_p = pad_gate_cols(w_ih.T)
    wcat = jnp.concatenate([whh_p, wih_p], axis=0)
    gate_scale = jnp.repeat(jnp.array([0.5, 0.5, 1.0, 0.5], jnp.float32), Hp)
    wcat = (wcat * gate_scale[None, :]).astype(jnp.bfloat16)
    bias = jnp.pad((b_ih + b_hh).astype(jnp.float32).reshape(4, H),
                   ((0, 0), (0, Hp - H))).reshape(1, 4 * Hp) * gate_scale[None, :]
    wout_p = jnp.pad(w_out.T, ((0, Hp - H), (0, Cp - C))).astype(jnp.bfloat16)
    bout_p = jnp.pad(b_out.astype(jnp.float32), (0, Cp - C)).reshape(1, Cp)

    emb_tm = jnp.take(emb_table, input_seq.T, axis=0).astype(jnp.bfloat16)
    emb_tm = jnp.pad(emb_tm, ((0, 0), (0, Bp - B), (0, 0)))

    while T % t_blk:
        t_blk //= 2
    grid = (T // t_blk,)

    out = pl.pallas_call(
        _lstm_fused_kernel,
        out_shape=jax.ShapeDtypeStruct((Bp, Cp), jnp.float32),
        grid_spec=pltpu.PrefetchScalarGridSpec(
            num_scalar_prefetch=0,
            grid=grid,
            in_specs=[
                pl.BlockSpec((t_blk, Bp, E), lambda t: (t, 0, 0)),
                pl.BlockSpec((K, 4 * Hp), lambda t: (0, 0)),
                pl.BlockSpec((1, 4 * Hp), lambda t: (0, 0)),
                pl.BlockSpec((Hp, Cp), lambda t: (0, 0)),
                pl.BlockSpec((1, Cp), lambda t: (0, 0)),
            ],
            out_specs=pl.BlockSpec((Bp, Cp), lambda t: (0, 0)),
            scratch_shapes=[
                pltpu.VMEM((Bb, K), jnp.bfloat16),
                pltpu.VMEM((Bb, K), jnp.bfloat16),
                pltpu.VMEM((Bb, Hp), jnp.float32),
                pltpu.VMEM((Bb, Hp), jnp.float32),
                pltpu.VMEM((Bb, Hp), jnp.float32),
                pltpu.VMEM((Bb, Hp), jnp.float32),
            ],
        ),
        compiler_params=pltpu.CompilerParams(
            dimension_semantics=("arbitrary",),
            vmem_limit_bytes=100 * 1024 * 1024,
        ),
    )(emb_tm, wcat, bias, wout_p, bout_p)

    return out[:B, :C]


def kernel(input_seq, emb_table, w_ih, w_hh, b_ih, b_hh, w_out, b_out):
    return _forward(input_seq, emb_table, w_ih, w_hh, b_ih, b_hh,
                    w_out, b_out, t_blk=16)


# speedup vs baseline: 1.6093x; 1.3457x over previous
import functools

import jax
import jax.numpy as jnp
from jax.experimental import pallas as pl
from jax.experimental.pallas import tpu as pltpu


def _round_up(x, m):
    return ((x + m - 1) // m) * m


def _lstm_fused_kernel(emb_ref, wcat_ref, bias_ref, wout_ref, bout_ref,
                       out_ref,
                       hx0_scr, hx1_scr, c0_scr, c1_scr, m0_scr, m1_scr):
    t_blk_idx = pl.program_id(0)
    Hp = c0_scr.shape[1]
    Bb = c0_scr.shape[0]
    T_BLK = emb_ref.shape[0]

    @pl.when(t_blk_idx == 0)
    def _init():
        for scr in (hx0_scr, hx1_scr, c0_scr, c1_scr):
            scr[...] = jnp.zeros_like(scr)
        for scr in (m0_scr, m1_scr):
            scr[...] = jnp.full(scr.shape, -jnp.inf, scr.dtype)

    wcat = wcat_ref[...]
    bias = bias_ref[...]

    def step(t, half, hx_scr, c_scr, m_scr):
        hx_scr[:, Hp:] = emb_ref[t, half * Bb:(half + 1) * Bb].astype(
            hx_scr.dtype)
        gp = jnp.dot(hx_scr[...], wcat,
                     preferred_element_type=jnp.float32) + bias
        ti = jnp.tanh(gp[:, 0 * Hp:1 * Hp])
        tf = jnp.tanh(gp[:, 1 * Hp:2 * Hp])
        tg = jnp.tanh(gp[:, 2 * Hp:3 * Hp])
        to = jnp.tanh(gp[:, 3 * Hp:4 * Hp])
        c = c_scr[...]
        c_new = 0.5 * ((c + tg) + (tf * c + ti * tg))
        tc = jnp.tanh(c_new)
        h_new = 0.5 * (tc + to * tc)
        c_scr[...] = c_new
        m_scr[...] = jnp.maximum(m_scr[...], h_new)
        hx_scr[:, :Hp] = h_new.astype(hx_scr.dtype)

    for t in range(T_BLK):
        step(t, 0, hx0_scr, c0_scr, m0_scr)
        step(t, 1, hx1_scr, c1_scr, m1_scr)

    @pl.when(t_blk_idx == pl.num_programs(0) - 1)
    def _finalize():
        wout = wout_ref[...]
        out_ref[0:Bb] = (jnp.dot(m0_scr[...].astype(wout.dtype), wout,
                                 preferred_element_type=jnp.float32)
                         + bout_ref[...])
        out_ref[Bb:2 * Bb] = (jnp.dot(m1_scr[...].astype(wout.dtype), wout,
                                      preferred_element_type=jnp.float32)
                              + bout_ref[...])


@functools.partial(jax.jit, static_argnames=("t_blk",))
def _forward(input_seq, emb_table, w_ih, w_hh, b_ih, b_hh, w_out, b_out,
             *, t_blk=16):
    B, T = input_seq.shape
    E = emb_table.shape[1]
    H = w_hh.shape[1]
    C = w_out.shape[0]

    Bp = _round_up(B, 16)
    Bb = Bp // 2
    Hp = _round_up(H, 128)
    Cp = _round_up(C, 128)
    K = Hp + E

    def pad_gate_cols(w_t):
        k = w_t.shape[0]
        w4 = w_t.reshape(k, 4, H)
        w4 = jnp.pad(w4, ((0, 0), (0, 0), (0, Hp - H)))
        return w4.reshape(k, 4 * Hp)

    whh_p = jnp.pad(pad_gate_cols(w_hh.T), ((0, Hp - H), (0, 0)))
    wih_p = pad_gate_cols(w_ih.T)
    wcat = jnp.concatenate([whh_p, wih_p], axis=0)
    gate_scale = jnp.repeat(jnp.array([0.5, 0.5, 1.0, 0.5], jnp.float32), Hp)
    wcat = (wcat * gate_scale[None, :]).astype(jnp.bfloat16)
    bias = jnp.pad((b_ih + b_hh).astype(jnp.float32).reshape(4, H),
                   ((0, 0), (0, Hp - H))).reshape(1, 4 * Hp) * gate_scale[None, :]
    wout_p = jnp.pad(w_out.T, ((0, Hp - H), (0, Cp - C))).astype(jnp.bfloat16)
    bout_p = jnp.pad(b_out.astype(jnp.float32), (0, Cp - C)).reshape(1, Cp)

    emb_tm = jnp.take(emb_table, input_seq.T, axis=0)
    if Bp != B:
        emb_tm = jnp.pad(emb_tm, ((0, 0), (0, Bp - B), (0, 0)))

    while T % t_blk:
        t_blk //= 2
    grid = (T // t_blk,)

    out = pl.pallas_call(
        _lstm_fused_kernel,
        out_shape=jax.ShapeDtypeStruct((Bp, Cp), jnp.float32),
        grid_spec=pltpu.PrefetchScalarGridSpec(
            num_scalar_prefetch=0,
            grid=grid,
            in_specs=[
                pl.BlockSpec((t_blk, Bp, E), lambda t: (t, 0, 0)),
                pl.BlockSpec((K, 4 * Hp), lambda t: (0, 0)),
                pl.BlockSpec((1, 4 * Hp), lambda t: (0, 0)),
                pl.BlockSpec((Hp, Cp), lambda t: (0, 0)),
                pl.BlockSpec((1, Cp), lambda t: (0, 0)),
            ],
            out_specs=pl.BlockSpec((Bp, Cp), lambda t: (0, 0)),
            scratch_shapes=[
                pltpu.VMEM((Bb, K), jnp.bfloat16),
                pltpu.VMEM((Bb, K), jnp.bfloat16),
                pltpu.VMEM((Bb, Hp), jnp.float32),
                pltpu.VMEM((Bb, Hp), jnp.float32),
                pltpu.VMEM((Bb, Hp), jnp.float32),
                pltpu.VMEM((Bb, Hp), jnp.float32),
            ],
        ),
        compiler_params=pltpu.CompilerParams(
            dimension_semantics=("arbitrary",),
            vmem_limit_bytes=100 * 1024 * 1024,
        ),
    )(emb_tm, wcat, bias, wout_p, bout_p)

    return out[:B, :C]


def kernel(input_seq, emb_table, w_ih, w_hh, b_ih, b_hh, w_out, b_out):
    return _forward(input_seq, emb_table, w_ih, w_hh, b_ih, b_hh,
                    w_out, b_out, t_blk=32)


# speedup vs baseline: 2.0498x; 1.2737x over previous
import functools

import jax
import jax.numpy as jnp
from jax.experimental import pallas as pl
from jax.experimental.pallas import tpu as pltpu


def _round_up(x, m):
    return ((x + m - 1) // m) * m


def _lstm_fused_kernel(ids_ref, tab_ref, whh_ref, wih_ref, bias_ref,
                       wout_ref, bout_ref, out_ref,
                       xga0, xgb0, xga1, xgb1,
                       h0_scr, h1_scr, c0_scr, c1_scr, m0_scr, m1_scr):
    t_blk_idx = pl.program_id(0)
    Hp = c0_scr.shape[1]
    Bb = c0_scr.shape[0]
    Bp = 2 * Bb
    T_BLK = (ids_ref.shape[0] // Bp) // pl.num_programs(0)

    @pl.when(t_blk_idx == 0)
    def _init():
        for scr in (h0_scr, h1_scr, c0_scr, c1_scr):
            scr[...] = jnp.zeros_like(scr)
        for scr in (m0_scr, m1_scr):
            scr[...] = jnp.full(scr.shape, -jnp.inf, scr.dtype)

    whh = whh_ref[...]
    wih = wih_ref[...]
    bias = bias_ref[...]

    def gather(t, half, xg):
        base = t_blk_idx * (T_BLK * Bp) + t * Bp + half * Bb
        for r in range(Bb):
            idx = ids_ref[base + r]
            xg[pl.ds(r, 1), :] = tab_ref[pl.ds(idx, 1), :]

    def step(t, half, xg, h_bf, c_scr, m_scr):
        gather(t, half, xg)
        gx = jnp.dot(xg[...].astype(jnp.bfloat16), wih,
                     preferred_element_type=jnp.float32) + bias
        gp = gx + jnp.dot(h_bf, whh, preferred_element_type=jnp.float32)
        ti = jnp.tanh(gp[:, 0 * Hp:1 * Hp])
        tf = jnp.tanh(gp[:, 1 * Hp:2 * Hp])
        tg = jnp.tanh(gp[:, 2 * Hp:3 * Hp])
        to = jnp.tanh(gp[:, 3 * Hp:4 * Hp])
        c = c_scr[...]
        c_new = 0.5 * ((c + tg) + (tf * c + ti * tg))
        tc = jnp.tanh(c_new)
        h_new = 0.5 * (tc + to * tc)
        c_scr[...] = c_new
        m_scr[...] = jnp.maximum(m_scr[...], h_new)
        return h_new.astype(jnp.bfloat16)

    h0 = h0_scr[...]
    h1 = h1_scr[...]
    for t in range(T_BLK):
        h0 = step(t, 0, xga0 if t % 2 == 0 else xgb0, h0, c0_scr, m0_scr)
        h1 = step(t, 1, xga1 if t % 2 == 0 else xgb1, h1, c1_scr, m1_scr)
    h0_scr[...] = h0
    h1_scr[...] = h1

    @pl.when(t_blk_idx == pl.num_programs(0) - 1)
    def _finalize():
        wout = wout_ref[...]
        out_ref[0:Bb] = (jnp.dot(m0_scr[...].astype(wout.dtype), wout,
                                 preferred_element_type=jnp.float32)
                         + bout_ref[...])
        out_ref[Bb:2 * Bb] = (jnp.dot(m1_scr[...].astype(wout.dtype), wout,
                                      preferred_element_type=jnp.float32)
                              + bout_ref[...])


@functools.partial(jax.jit, static_argnames=("t_blk",))
def _forward(input_seq, emb_table, w_ih, w_hh, b_ih, b_hh, w_out, b_out,
             *, t_blk=16):
    B, T = input_seq.shape
    V, E = emb_table.shape
    H = w_hh.shape[1]
    C = w_out.shape[0]

    Bp = _round_up(B, 16)
    Bb = Bp // 2
    Hp = _round_up(H, 128)
    Cp = _round_up(C, 128)

    def pad_gate_cols(w_t):
        k = w_t.shape[0]
        w4 = w_t.reshape(k, 4, H)
        w4 = jnp.pad(w4, ((0, 0), (0, 0), (0, Hp - H)))
        return w4.reshape(k, 4 * Hp)

    gate_scale = jnp.repeat(jnp.array([0.5, 0.5, 1.0, 0.5], jnp.float32), Hp)
    whh_p = (jnp.pad(pad_gate_cols(w_hh.T), ((0, Hp - H), (0, 0)))
             * gate_scale[None, :]).astype(jnp.bfloat16)
    wih_p = (pad_gate_cols(w_ih.T)
             * gate_scale[None, :]).astype(jnp.bfloat16)
    bias = jnp.pad((b_ih + b_hh).astype(jnp.float32).reshape(4, H),
                   ((0, 0), (0, Hp - H))).reshape(1, 4 * Hp) * gate_scale[None, :]
    wout_p = jnp.pad(w_out.T, ((0, Hp - H), (0, Cp - C))).astype(jnp.bfloat16)
    bout_p = jnp.pad(b_out.astype(jnp.float32), (0, Cp - C)).reshape(1, Cp)

    ids = input_seq.T
    if Bp != B:
        ids = jnp.pad(ids, ((0, 0), (0, Bp - B)))
    ids = ids.reshape(T * Bp)

    while T % t_blk:
        t_blk //= 2
    grid = (T // t_blk,)

    out = pl.pallas_call(
        _lstm_fused_kernel,
        out_shape=jax.ShapeDtypeStruct((Bp, Cp), jnp.float32),
        grid_spec=pltpu.PrefetchScalarGridSpec(
            num_scalar_prefetch=1,
            grid=grid,
            in_specs=[
                pl.BlockSpec((V, E), lambda t, ids_r: (0, 0)),
                pl.BlockSpec((Hp, 4 * Hp), lambda t, ids_r: (0, 0)),
                pl.BlockSpec((E, 4 * Hp), lambda t, ids_r: (0, 0)),
                pl.BlockSpec((1, 4 * Hp), lambda t, ids_r: (0, 0)),
                pl.BlockSpec((Hp, Cp), lambda t, ids_r: (0, 0)),
                pl.BlockSpec((1, Cp), lambda t, ids_r: (0, 0)),
            ],
            out_specs=pl.BlockSpec((Bp, Cp), lambda t, ids_r: (0, 0)),
            scratch_shapes=[
                pltpu.VMEM((Bb, E), jnp.float32),
                pltpu.VMEM((Bb, E), jnp.float32),
                pltpu.VMEM((Bb, E), jnp.float32),
                pltpu.VMEM((Bb, E), jnp.float32),
                pltpu.VMEM((Bb, Hp), jnp.bfloat16),
                pltpu.VMEM((Bb, Hp), jnp.bfloat16),
                pltpu.VMEM((Bb, Hp), jnp.float32),
                pltpu.VMEM((Bb, Hp), jnp.float32),
                pltpu.VMEM((Bb, Hp), jnp.float32),
                pltpu.VMEM((Bb, Hp), jnp.float32),
            ],
        ),
        compiler_params=pltpu.CompilerParams(
            dimension_semantics=("arbitrary",),
            vmem_limit_bytes=100 * 1024 * 1024,
        ),
    )(ids, emb_table, whh_p, wih_p, bias, wout_p, bout_p)

    return out[:B, :C]


def kernel(input_seq, emb_table, w_ih, w_hh, b_ih, b_hh, w_out, b_out):
    return _forward(input_seq, emb_table, w_ih, w_hh, b_ih, b_hh,
                    w_out, b_out, t_blk=16)


# speedup vs baseline: 2.0545x; 1.0023x over previous
import functools

import jax
import jax.numpy as jnp
from jax.experimental import pallas as pl
from jax.experimental.pallas import tpu as pltpu


def _round_up(x, m):
    return ((x + m - 1) // m) * m


def _lstm_fused_kernel(ids_ref, tab_ref, whh_ref, wih_ref, bias_ref,
                       wout_ref, bout_ref, out_ref,
                       xga0, xgb0, xga1, xgb1,
                       h0_scr, h1_scr, c0_scr, c1_scr, m0_scr, m1_scr):
    t_blk_idx = pl.program_id(0)
    Hp = c0_scr.shape[1]
    Bb = c0_scr.shape[0]
    Bp = 2 * Bb
    T_BLK = (ids_ref.shape[0] // Bp) // pl.num_programs(0)

    @pl.when(t_blk_idx == 0)
    def _init():
        for scr in (h0_scr, h1_scr, c0_scr, c1_scr):
            scr[...] = jnp.zeros_like(scr)
        for scr in (m0_scr, m1_scr):
            scr[...] = jnp.full(scr.shape, -jnp.inf, scr.dtype)

    whh = whh_ref[...]
    wih = wih_ref[...]
    bias = bias_ref[...]

    def gather(t, half, xg):
        base = t_blk_idx * (T_BLK * Bp) + t * Bp + half * Bb
        for r in range(Bb):
            idx = ids_ref[base + r]
            xg[pl.ds(r, 1), :] = tab_ref[pl.ds(idx, 1), :]

    def step(t, half, xg, h_bf, c_scr, m_scr):
        gather(t, half, xg)
        gx = jnp.dot(xg[...].astype(jnp.bfloat16), wih,
                     preferred_element_type=jnp.float32) + bias
        gp = gx + jnp.dot(h_bf, whh, preferred_element_type=jnp.float32)
        ti = jnp.tanh(gp[:, 0 * Hp:1 * Hp])
        tf = jnp.tanh(gp[:, 1 * Hp:2 * Hp])
        tg = jnp.tanh(gp[:, 2 * Hp:3 * Hp])
        to = jnp.tanh(gp[:, 3 * Hp:4 * Hp])
        c = c_scr[...]
        c_new = 0.5 * ((c + tg) + (tf * c + ti * tg))
        tc = jnp.tanh(c_new)
        h_new = 0.5 * (tc + to * tc)
        c_scr[...] = c_new
        m_scr[...] = jnp.maximum(m_scr[...], h_new)
        return h_new.astype(jnp.bfloat16)

    h0 = h0_scr[...]
    h1 = h1_scr[...]
    for t in range(T_BLK):
        h0 = step(t, 0, xga0 if t % 2 == 0 else xgb0, h0, c0_scr, m0_scr)
        h1 = step(t, 1, xga1 if t % 2 == 0 else xgb1, h1, c1_scr, m1_scr)
    h0_scr[...] = h0
    h1_scr[...] = h1

    @pl.when(t_blk_idx == pl.num_programs(0) - 1)
    def _finalize():
        wout = wout_ref[...]
        out_ref[0:Bb] = (jnp.dot(m0_scr[...].astype(wout.dtype), wout,
                                 preferred_element_type=jnp.float32)
                         + bout_ref[...])
        out_ref[Bb:2 * Bb] = (jnp.dot(m1_scr[...].astype(wout.dtype), wout,
                                      preferred_element_type=jnp.float32)
                              + bout_ref[...])


@functools.partial(jax.jit, static_argnames=("t_blk",))
def _forward(input_seq, emb_table, w_ih, w_hh, b_ih, b_hh, w_out, b_out,
             *, t_blk=16):
    B, T = input_seq.shape
    V, E = emb_table.shape
    H = w_hh.shape[1]
    C = w_out.shape[0]

    Bp = _round_up(B, 16)
    Bb = Bp // 2
    Hp = _round_up(H, 128)
    Cp = _round_up(C, 128)

    def pad_gate_cols(w_t):
        k = w_t.shape[0]
        w4 = w_t.reshape(k, 4, H)
        w4 = jnp.pad(w4, ((0, 0), (0, 0), (0, Hp - H)))
        return w4.reshape(k, 4 * Hp)

    gate_scale = jnp.repeat(jnp.array([0.5, 0.5, 1.0, 0.5], jnp.float32), Hp)
    whh_p = (jnp.pad(pad_gate_cols(w_hh.T), ((0, Hp - H), (0, 0)))
             * gate_scale[None, :]).astype(jnp.bfloat16)
    wih_p = (pad_gate_cols(w_ih.T)
             * gate_scale[None, :]).astype(jnp.bfloat16)
    bias = jnp.pad((b_ih + b_hh).astype(jnp.float32).reshape(4, H),
                   ((0, 0), (0, Hp - H))).reshape(1, 4 * Hp) * gate_scale[None, :]
    wout_p = jnp.pad(w_out.T, ((0, Hp - H), (0, Cp - C))).astype(jnp.bfloat16)
    bout_p = jnp.pad(b_out.astype(jnp.float32), (0, Cp - C)).reshape(1, Cp)

    ids = input_seq.T
    if Bp != B:
        ids = jnp.pad(ids, ((0, 0), (0, Bp - B)))
    ids = ids.reshape(T * Bp)

    while T % t_blk:
        t_blk //= 2
    grid = (T // t_blk,)

    out = pl.pallas_call(
        _lstm_fused_kernel,
        out_shape=jax.ShapeDtypeStruct((Bp, Cp), jnp.float32),
        grid_spec=pltpu.PrefetchScalarGridSpec(
            num_scalar_prefetch=1,
            grid=grid,
            in_specs=[
                pl.BlockSpec((V, E), lambda t, ids_r: (0, 0)),
                pl.BlockSpec((Hp, 4 * Hp), lambda t, ids_r: (0, 0)),
                pl.BlockSpec((E, 4 * Hp), lambda t, ids_r: (0, 0)),
                pl.BlockSpec((1, 4 * Hp), lambda t, ids_r: (0, 0)),
                pl.BlockSpec((Hp, Cp), lambda t, ids_r: (0, 0)),
                pl.BlockSpec((1, Cp), lambda t, ids_r: (0, 0)),
            ],
            out_specs=pl.BlockSpec((Bp, Cp), lambda t, ids_r: (0, 0)),
            scratch_shapes=[
                pltpu.VMEM((Bb, E), jnp.float32),
                pltpu.VMEM((Bb, E), jnp.float32),
                pltpu.VMEM((Bb, E), jnp.float32),
                pltpu.VMEM((Bb, E), jnp.float32),
                pltpu.VMEM((Bb, Hp), jnp.bfloat16),
                pltpu.VMEM((Bb, Hp), jnp.bfloat16),
                pltpu.VMEM((Bb, Hp), jnp.float32),
                pltpu.VMEM((Bb, Hp), jnp.float32),
                pltpu.VMEM((Bb, Hp), jnp.float32),
                pltpu.VMEM((Bb, Hp), jnp.float32),
            ],
        ),
        compiler_params=pltpu.CompilerParams(
            dimension_semantics=("arbitrary",),
            vmem_limit_bytes=100 * 1024 * 1024,
        ),
    )(ids, emb_table, whh_p, wih_p, bias, wout_p, bout_p)

    return out[:B, :C]


def kernel(input_seq, emb_table, w_ih, w_hh, b_ih, b_hh, w_out, b_out):
    return _forward(input_seq, emb_table, w_ih, w_hh, b_ih, b_hh,
                    w_out, b_out, t_blk=32)


# speedup vs baseline: 2.0944x; 1.0194x over previous
import functools

import jax
import jax.numpy as jnp
from jax.experimental import pallas as pl
from jax.experimental.pallas import tpu as pltpu


def _round_up(x, m):
    return ((x + m - 1) // m) * m


def _lstm_fused_kernel(ids_ref, tab_ref, wcat_ref, bias_ref,
                       wout_ref, bout_ref, out_ref,
                       xga0, xgb0, xga1, xgb1,
                       h0_scr, h1_scr, c0_scr, c1_scr, m0_scr, m1_scr):
    t_blk_idx = pl.program_id(0)
    Hp = c0_scr.shape[1]
    Bb = c0_scr.shape[0]
    Bp = 2 * Bb
    T_BLK = (ids_ref.shape[0] // Bp) // pl.num_programs(0)

    @pl.when(t_blk_idx == 0)
    def _init():
        for scr in (h0_scr, h1_scr, c0_scr, c1_scr):
            scr[...] = jnp.zeros_like(scr)
        for scr in (m0_scr, m1_scr):
            scr[...] = jnp.full(scr.shape, -jnp.inf, scr.dtype)

    wcat = wcat_ref[...]
    bias = bias_ref[...]

    def gather(t, half, xg):
        base = t_blk_idx * (T_BLK * Bp) + t * Bp + half * Bb
        for r in range(Bb):
            idx = ids_ref[base + r]
            xg[pl.ds(r, 1), :] = tab_ref[pl.ds(idx, 1), :]

    def step(t, half, xg, h_bf, c_scr, m_scr):
        gather(t, half, xg)
        hx = jnp.concatenate([h_bf, xg[...].astype(jnp.bfloat16)], axis=1)
        gp = jnp.dot(hx, wcat, preferred_element_type=jnp.float32) + bias
        ti = jnp.tanh(gp[:, 0 * Hp:1 * Hp])
        tf = jnp.tanh(gp[:, 1 * Hp:2 * Hp])
        tg = jnp.tanh(gp[:, 2 * Hp:3 * Hp])
        to = jnp.tanh(gp[:, 3 * Hp:4 * Hp])
        c = c_scr[...]
        c_new = 0.5 * ((c + tg) + (tf * c + ti * tg))
        tc = jnp.tanh(c_new)
        h_new = 0.5 * (tc + to * tc)
        c_scr[...] = c_new
        m_scr[...] = jnp.maximum(m_scr[...], h_new)
        return h_new.astype(jnp.bfloat16)

    h0 = h0_scr[...]
    h1 = h1_scr[...]
    for t in range(T_BLK):
        h0 = step(t, 0, xga0 if t % 2 == 0 else xgb0, h0, c0_scr, m0_scr)
        h1 = step(t, 1, xga1 if t % 2 == 0 else xgb1, h1, c1_scr, m1_scr)
    h0_scr[...] = h0
    h1_scr[...] = h1

    @pl.when(t_blk_idx == pl.num_programs(0) - 1)
    def _finalize():
        wout = wout_ref[...]
        out_ref[0:Bb] = (jnp.dot(m0_scr[...].astype(wout.dtype), wout,
                                 preferred_element_type=jnp.float32)
                         + bout_ref[...])
        out_ref[Bb:2 * Bb] = (jnp.dot(m1_scr[...].astype(wout.dtype), wout,
                                      preferred_element_type=jnp.float32)
                              + bout_ref[...])


@functools.partial(jax.jit, static_argnames=("t_blk",))
def _forward(input_seq, emb_table, w_ih, w_hh, b_ih, b_hh, w_out, b_out,
             *, t_blk=16):
    B, T = input_seq.shape
    V, E = emb_table.shape
    H = w_hh.shape[1]
    C = w_out.shape[0]

    Bp = _round_up(B, 16)
    Bb = Bp // 2
    Hp = _round_up(H, 128)
    Cp = _round_up(C, 128)

    def pad_gate_cols(w_t):
        k = w_t.shape[0]
        w4 = w_t.reshape(k, 4, H)
        w4 = jnp.pad(w4, ((0, 0), (0, 0), (0, Hp - H)))
        return w4.reshape(k, 4 * Hp)

    gate_scale = jnp.repeat(jnp.array([0.5, 0.5, 1.0, 0.5], jnp.float32), Hp)
    whh_p = jnp.pad(pad_gate_cols(w_hh.T), ((0, Hp - H), (0, 0)))
    wih_p = pad_gate_cols(w_ih.T)
    wcat = (jnp.concatenate([whh_p, wih_p], axis=0)
            * gate_scale[None, :]).astype(jnp.bfloat16)
    bias = jnp.pad((b_ih + b_hh).astype(jnp.float32).reshape(4, H),
                   ((0, 0), (0, Hp - H))).reshape(1, 4 * Hp) * gate_scale[None, :]
    wout_p = jnp.pad(w_out.T, ((0, Hp - H), (0, Cp - C))).astype(jnp.bfloat16)
    bout_p = jnp.pad(b_out.astype(jnp.float32), (0, Cp - C)).reshape(1, Cp)

    ids = input_seq.T
    if Bp != B:
        ids = jnp.pad(ids, ((0, 0), (0, Bp - B)))
    ids = ids.reshape(T * Bp)

    while T % t_blk:
        t_blk //= 2
    grid = (T // t_blk,)

    out = pl.pallas_call(
        _lstm_fused_kernel,
        out_shape=jax.ShapeDtypeStruct((Bp, Cp), jnp.float32),
        grid_spec=pltpu.PrefetchScalarGridSpec(
            num_scalar_prefetch=1,
            grid=grid,
            in_specs=[
                pl.BlockSpec((V, E), lambda t, ids_r: (0, 0)),
                pl.BlockSpec((Hp + E, 4 * Hp), lambda t, ids_r: (0, 0)),
                pl.BlockSpec((1, 4 * Hp), lambda t, ids_r: (0, 0)),
                pl.BlockSpec((Hp, Cp), lambda t, ids_r: (0, 0)),
                pl.BlockSpec((1, Cp), lambda t, ids_r: (0, 0)),
            ],
            out_specs=pl.BlockSpec((Bp, Cp), lambda t, ids_r: (0, 0)),
            scratch_shapes=[
                pltpu.VMEM((Bb, E), jnp.float32),
                pltpu.VMEM((Bb, E), jnp.float32),
                pltpu.VMEM((Bb, E), jnp.float32),
                pltpu.VMEM((Bb, E), jnp.float32),
                pltpu.VMEM((Bb, Hp), jnp.bfloat16),
                pltpu.VMEM((Bb, Hp), jnp.bfloat16),
                pltpu.VMEM((Bb, Hp), jnp.float32),
                pltpu.VMEM((Bb, Hp), jnp.float32),
                pltpu.VMEM((Bb, Hp), jnp.float32),
                pltpu.VMEM((Bb, Hp), jnp.float32),
            ],
        ),
        compiler_params=pltpu.CompilerParams(
            dimension_semantics=("arbitrary",),
            vmem_limit_bytes=100 * 1024 * 1024,
        ),
    )(ids, emb_table, wcat, bias, wout_p, bout_p)

    return out[:B, :C]


def kernel(input_seq, emb_table, w_ih, w_hh, b_ih, b_hh, w_out, b_out):
    return _forward(input_seq, emb_table, w_ih, w_hh, b_ih, b_hh,
                    w_out, b_out, t_blk=32)
